# baseline (speedup 1.0000x reference)
import functools

import jax
import jax.numpy as jnp
from jax import lax
from jax.experimental import pallas as pl
from jax.experimental.pallas import tpu as pltpu
from jax.experimental.pallas import tpu_sc as plsc


@functools.lru_cache(maxsize=None)
def _make_sc_gather(n_rows: int, emb_dim: int):
    info = plsc.get_sparse_core_info()
    nc, ns, lanes = info.num_cores, info.num_subcores, info.num_lanes
    nw = nc * ns
    rows_per_w = n_rows // nw
    chunk = 128
    n_chunk = rows_per_w // chunk
    assert rows_per_w % chunk == 0 and n_rows % nw == 0

    mesh = plsc.VectorSubcoreMesh(core_axis_name="c", subcore_axis_name="s")

    @functools.partial(
        pl.kernel,
        mesh=mesh,
        out_type=jax.ShapeDtypeStruct((n_rows, emb_dim), jnp.float32),
        scratch_types=[
            pltpu.VMEM((n_chunk, chunk), jnp.int32),
            pltpu.VMEM((n_chunk, chunk), jnp.int32),
            pltpu.VMEM((chunk, 2 * emb_dim), jnp.float32),
            pltpu.VMEM((chunk, 2 * emb_dim), jnp.float32),
            pltpu.VMEM((rows_per_w, emb_dim), jnp.float32),
            pltpu.SemaphoreType.DMA,
            pltpu.SemaphoreType.DMA,
        ],
        compiler_params=pltpu.CompilerParams(needs_layout_passes=False),
    )
    def gather_k(tidx_hbm, sub_hbm, table_hbm, out_hbm,
                 tidx_v, sub_v, buf_a, buf_b, out_v, sem_a, sem_b):
        wid = lax.axis_index("s") * nc + lax.axis_index("c")
        pltpu.sync_copy(tidx_hbm.at[wid], tidx_v)
        pltpu.sync_copy(sub_hbm.at[wid], sub_v)
        lane_iota = lax.iota(jnp.int32, lanes)
        bufs = [buf_a, buf_b]
        sems = [sem_a, sem_b]

        def issue(k):
            return pltpu.async_copy(table_hbm.at[tidx_v.at[k]],
                                    bufs[k % 2], sems[k % 2])

        def extract(k):
            for g in range(chunk // lanes):
                tok16 = lane_iota + g * lanes
                off16 = sub_v[k, pl.ds(g * lanes, lanes)]
                dst16 = tok16 + k * chunk

                def col_body(ci, _):
                    for u in range(4):
                        c16 = jnp.full((lanes,), ci * 4 + u, jnp.int32)
                        vals = plsc.load_gather(bufs[k % 2],
                                                [tok16, off16 + c16])
                        plsc.store_scatter(out_v, [dst16, c16], vals)
                    return 0

                lax.fori_loop(0, emb_dim // 4, col_body, 0)

        copies = [issue(0)]
        for k in range(n_chunk):
            if k + 1 < n_chunk:
                copies.append(issue(k + 1))
            copies[k].wait()
            extract(k)
        pltpu.sync_copy(out_v, out_hbm.at[pl.ds(wid * rows_per_w, rows_per_w)])

    return gather_k


_CB = 1024


def _conv_body(a_ref, b_ref, eye_ref, o_ref):
    E = a_ref.shape[0]
    eye = eye_ref[...]
    dn = (((0,), (0,)), ((), ()))
    o_ref[:, 0:E] = lax.dot_general(a_ref[...], eye, dn,
                                    preferred_element_type=jnp.float32)
    o_ref[:, E:2 * E] = lax.dot_general(b_ref[...], eye, dn,
                                        preferred_element_type=jnp.float32)


def _compact(tT, table):
    E, V = tT.shape
    nb = V // (2 * _CB)
    main_rows = nb * _CB
    eye = jnp.eye(E, dtype=jnp.float32)
    n_tail = V - 2 * main_rows
    conv = pl.pallas_call(
        _conv_body,
        grid=(nb,),
        in_specs=[
            pl.BlockSpec((E, _CB), lambda i: (0, 2 * i)),
            pl.BlockSpec((E, _CB), lambda i: (0, 2 * i + 1)),
            pl.BlockSpec((E, E), lambda i: (0, 0)),
        ],
        out_specs=pl.BlockSpec((_CB, 2 * E), lambda i: (i, 0)),
        out_shape=jax.ShapeDtypeStruct((main_rows + n_tail, 2 * E),
                                       jnp.float32),
    )(tT, tT, eye)
    tail = lax.slice(table, (2 * main_rows, 0), (V, E))
    return lax.dynamic_update_slice(conv, tail, (main_rows, 0))


def _lstm_body(L, H, emb_ref, mask_ref, wih_ref, whh_ref, b_ref,
               h_out, c_out, h_s, c_s):
    t = pl.program_id(0)

    @pl.when(t == 0)
    def _init():
        h_s[...] = jnp.zeros_like(h_s)
        c_s[...] = jnp.zeros_like(c_s)

    xt = emb_ref[0] * mask_ref[0]
    h = h_s[...]
    c = c_s[...]
    gates = lax.dot_general(xt, wih_ref[...], (((1,), (1,)), ((), ())),
                            preferred_element_type=jnp.float32)
    gates = gates + lax.dot_general(h, whh_ref[...], (((1,), (1,)), ((), ())),
                                    preferred_element_type=jnp.float32)
    gates = gates + b_ref[...]
    i = jax.nn.sigmoid(gates[:, 0:H])
    f = jax.nn.sigmoid(gates[:, H:2 * H])
    g = jnp.tanh(gates[:, 2 * H:3 * H])
    o = jax.nn.sigmoid(gates[:, 3 * H:4 * H])
    c_new = f * c + i * g
    h_new = o * jnp.tanh(c_new)
    h_s[...] = h_new
    c_s[...] = c_new

    @pl.when(t == L - 1)
    def _emit():
        h_out[...] = h_new
        c_out[...] = c_new


def _lstm(embT, mask3, W_ih, W_hh, b2):
    L, B, E = embT.shape
    H = W_hh.shape[1]
    return pl.pallas_call(
        functools.partial(_lstm_body, L, H),
        grid=(L,),
        in_specs=[
            pl.BlockSpec((1, B, E), lambda t: (t, 0, 0)),
            pl.BlockSpec((1, B, 1), lambda t: (t, 0, 0)),
            pl.BlockSpec((4 * H, E), lambda t: (0, 0)),
            pl.BlockSpec((4 * H, H), lambda t: (0, 0)),
            pl.BlockSpec((1, 4 * H), lambda t: (0, 0)),
        ],
        out_specs=[
            pl.BlockSpec((B, H), lambda t: (0, 0)),
            pl.BlockSpec((B, H), lambda t: (0, 0)),
        ],
        out_shape=[jax.ShapeDtypeStruct((B, H), jnp.float32)] * 2,
        scratch_shapes=[
            pltpu.VMEM((B, H), jnp.float32),
            pltpu.VMEM((B, H), jnp.float32),
        ],
    )(embT, mask3, W_ih, W_hh, b2)


def kernel(x, table, W_ih, W_hh, b_ih, b_hh):
    B, L = x.shape
    V, E = table.shape
    H = W_hh.shape[1]
    nw, chunk = 32, 128

    xT = jnp.transpose(x)
    flat_idx = xT.reshape(-1)
    tidx = (((flat_idx >> 11) << 10) + (flat_idx & 1023)).reshape(
        nw, -1, chunk)
    sub = (((flat_idx >> 10) & 1) * E).reshape(nw, -1, chunk)
    tT = jnp.transpose(table)
    table2 = _compact(tT, table)

    emb_flat = _make_sc_gather(L * B, E)(tidx, sub, table2)
    embT = emb_flat.reshape(L, B, E)
    mask3 = (xT != 0).astype(jnp.float32).reshape(L, B, 1)
    b2 = (b_ih + b_hh).reshape(1, 4 * H)

    hN, cN = _lstm(embT, mask3, W_ih, W_hh, b2)
    return hN[None, :, :], cN[None, :, :]
